# Initial kernel scaffold; baseline (speedup 1.0000x reference)
import jax
import jax.numpy as jnp
from jax.experimental import pallas as pl


def kernel(grad_output, input, target, total_weight):
    raise NotImplementedError("write your pallas kernel here")


# speedup vs baseline: 3.2676x; 3.2676x over previous
import jax
import jax.numpy as jnp
from jax import lax
from jax.experimental import pallas as pl
from jax.experimental.pallas import tpu as pltpu
from jax.experimental.pallas import tpu_sc as plsc

IGNORE = 1
L = 16
NC, NS = 2, 16
NW = NC * NS


def _make_sc_kernel(N, C):
    RPW = N // NW
    ZROWS = 8
    ZLEN = ZROWS * C
    NCHUNK = RPW // ZROWS
    HALF = RPW // 2

    def body(g_hbm, t_hbm, out_hbm, zbuf, tloc, gloc,
             idx_a, idx_b, val_a, val_b, sem):
        wid = lax.axis_index("s") * NC + lax.axis_index("c")
        base = wid * RPW

        pltpu.sync_copy(t_hbm.at[pl.ds(base, RPW)], tloc)
        pltpu.sync_copy(g_hbm.at[pl.ds(base, RPW)], gloc)

        zeros16 = jnp.zeros((L,), jnp.float32)

        def zinit(i, carry):
            zbuf[pl.ds(i * L, L)] = zeros16
            return carry

        lax.fori_loop(0, ZLEN // L, zinit, 0)

        iota16 = lax.iota(jnp.int32, L)
        for j in range(RPW // L):
            t16 = tloc[pl.ds(j * L, L)]
            g16 = gloc[pl.ds(j * L, L)]
            rows = (base + j * L) + iota16
            idx16 = rows * C + t16
            val16 = jnp.where(t16 == IGNORE, zeros16, -g16)
            if j < HALF // L:
                idx_a[pl.ds(j * L, L)] = idx16
                val_a[pl.ds(j * L, L)] = val16
            else:
                idx_b[pl.ds(j * L - HALF, L)] = idx16
                val_b[pl.ds(j * L - HALF, L)] = val16

        def zfill(c, carry):
            start = pl.multiple_of(base * C + c * ZLEN, ZLEN)
            pltpu.sync_copy(zbuf, out_hbm.at[pl.ds(start, ZLEN)])
            return carry

        lax.fori_loop(0, NCHUNK, zfill, 0)

        pltpu.async_copy(val_a, out_hbm.at[idx_a], sem).wait()
        pltpu.async_copy(val_b, out_hbm.at[idx_b], sem).wait()

    mesh = plsc.VectorSubcoreMesh(core_axis_name="c", subcore_axis_name="s")
    return pl.kernel(
        body,
        out_type=jax.ShapeDtypeStruct((N * C,), jnp.float32),
        mesh=mesh,
        scratch_types=[
            pltpu.VMEM((ZLEN,), jnp.float32),
            pltpu.VMEM((RPW,), jnp.int32),
            pltpu.VMEM((RPW,), jnp.float32),
            pltpu.VMEM((HALF,), jnp.int32),
            pltpu.VMEM((HALF,), jnp.int32),
            pltpu.VMEM((HALF,), jnp.float32),
            pltpu.VMEM((HALF,), jnp.float32),
            pltpu.SemaphoreType.DMA,
        ],
    )


def kernel(grad_output, input, target, total_weight):
    N, C = input.shape
    tgt = target.astype(jnp.int32)
    out = _make_sc_kernel(N, C)(grad_output, tgt)
    return out.reshape(N, C)


# speedup vs baseline: 3.4176x; 1.0459x over previous
import jax
import jax.numpy as jnp
from jax import lax
from jax.experimental import pallas as pl
from jax.experimental.pallas import tpu as pltpu
from jax.experimental.pallas import tpu_sc as plsc

IGNORE = 1
L = 16
NC, NS = 2, 16
NW = NC * NS


def _make_sc_kernel(N, C):
    RPW = N // NW
    ZROWS = 8
    ZLEN = ZROWS * C
    NCHUNK = RPW // ZROWS
    HALF = RPW // 2

    def body(g_hbm, t_hbm, out_hbm, zbuf, tloc, gloc,
             idx_a, idx_b, val_a, val_b, sem):
        wid = lax.axis_index("s") * NC + lax.axis_index("c")
        base = wid * RPW

        pltpu.sync_copy(t_hbm.at[pl.ds(base, RPW)], tloc)
        pltpu.sync_copy(g_hbm.at[pl.ds(base, RPW)], gloc)

        zeros16 = jnp.zeros((L,), jnp.float32)
        UNROLL = 8

        def zinit(i, carry):
            for u in range(UNROLL):
                zbuf[pl.ds(i * (L * UNROLL) + u * L, L)] = zeros16
            return carry

        lax.fori_loop(0, ZLEN // (L * UNROLL), zinit, 0)

        def zfire(c, carry):
            start = pl.multiple_of(base * C + c * ZLEN, ZLEN)
            pltpu.make_async_copy(zbuf, out_hbm.at[pl.ds(start, ZLEN)], sem).start()
            return carry

        lax.fori_loop(0, NCHUNK, zfire, 0)

        iota16 = lax.iota(jnp.int32, L)
        for j in range(RPW // L):
            t16 = tloc[pl.ds(j * L, L)]
            g16 = gloc[pl.ds(j * L, L)]
            rows = (base + j * L) + iota16
            idx16 = rows * C + t16
            val16 = jnp.where(t16 == IGNORE, zeros16, -g16)
            if j < HALF // L:
                idx_a[pl.ds(j * L, L)] = idx16
                val_a[pl.ds(j * L, L)] = val16
            else:
                idx_b[pl.ds(j * L - HALF, L)] = idx16
                val_b[pl.ds(j * L - HALF, L)] = val16

        def zdrain(c, carry):
            start = pl.multiple_of(base * C + c * ZLEN, ZLEN)
            pltpu.make_async_copy(zbuf, out_hbm.at[pl.ds(start, ZLEN)], sem).wait()
            return carry

        lax.fori_loop(0, NCHUNK, zdrain, 0)

        pltpu.async_copy(val_a, out_hbm.at[idx_a], sem).wait()
        pltpu.async_copy(val_b, out_hbm.at[idx_b], sem).wait()

    mesh = plsc.VectorSubcoreMesh(core_axis_name="c", subcore_axis_name="s")
    return pl.kernel(
        body,
        out_type=jax.ShapeDtypeStruct((N * C,), jnp.float32),
        mesh=mesh,
        scratch_types=[
            pltpu.VMEM((ZLEN,), jnp.float32),
            pltpu.VMEM((RPW,), jnp.int32),
            pltpu.VMEM((RPW,), jnp.float32),
            pltpu.VMEM((HALF,), jnp.int32),
            pltpu.VMEM((HALF,), jnp.int32),
            pltpu.VMEM((HALF,), jnp.float32),
            pltpu.VMEM((HALF,), jnp.float32),
            pltpu.SemaphoreType.DMA,
        ],
    )


def kernel(grad_output, input, target, total_weight):
    N, C = input.shape
    tgt = target.astype(jnp.int32)
    out = _make_sc_kernel(N, C)(grad_output, tgt)
    return out.reshape(N, C)


# speedup vs baseline: 12.1423x; 3.5529x over previous
import jax
import jax.numpy as jnp
from jax import lax
from jax.experimental import pallas as pl
from jax.experimental.pallas import tpu as pltpu
from jax.experimental.pallas import tpu_sc as plsc

IGNORE = 1
L = 16
NC, NS = 2, 16
NW = NC * NS


def _make_sc_kernel(N, C):
    RPW = N // NW
    ZROWS = 4
    NCHUNK = RPW // ZROWS
    GRP = L // ZROWS

    def chunk_vectors(tloc, gloc, iota16, zeros16, c):
        grp, sub = divmod(c, GRP)
        t16 = tloc[pl.ds(grp * L, L)]
        g16 = gloc[pl.ds(grp * L, L)]
        val16 = jnp.where(t16 == IGNORE, zeros16, -g16)
        ridx16 = iota16 - (ZROWS * sub)
        mask16 = (iota16 >= ZROWS * sub) & (iota16 < ZROWS * (sub + 1))
        return t16, val16, ridx16, mask16

    def body(g_hbm, t_hbm, out_hbm, buf0, buf1, tloc, gloc, sem0, sem1):
        wid = lax.axis_index("s") * NC + lax.axis_index("c")
        base = wid * RPW

        pltpu.sync_copy(t_hbm.at[pl.ds(base, RPW)], tloc)
        pltpu.sync_copy(g_hbm.at[pl.ds(base, RPW)], gloc)

        zeros16 = jnp.zeros((L,), jnp.float32)
        iota16 = lax.iota(jnp.int32, L)
        bufs = (buf0, buf1)
        sems = (sem0, sem1)
        UNROLL = 8

        def zinit(i, carry):
            for r in range(ZROWS):
                for u in range(UNROLL):
                    off = pl.multiple_of(i * (L * UNROLL), L * UNROLL) + u * L
                    buf0[r, pl.ds(off, L)] = zeros16
                    buf1[r, pl.ds(off, L)] = zeros16
            return carry

        lax.fori_loop(0, C // (L * UNROLL), zinit, 0)

        for c in range(NCHUNK):
            b, sem = bufs[c % 2], sems[c % 2]
            if c >= 2:
                pc = c - 2
                prow = pl.multiple_of(base + pc * ZROWS, ZROWS)
                pltpu.make_async_copy(b, out_hbm.at[pl.ds(prow, ZROWS)], sem).wait()
                pt16, _, pr16, pm16 = chunk_vectors(tloc, gloc, iota16, zeros16, pc)
                plsc.store_scatter(b, [pr16, pt16], zeros16, mask=pm16)
            t16, val16, ridx16, mask16 = chunk_vectors(tloc, gloc, iota16, zeros16, c)
            plsc.store_scatter(b, [ridx16, t16], val16, mask=mask16)
            row = pl.multiple_of(base + c * ZROWS, ZROWS)
            pltpu.make_async_copy(b, out_hbm.at[pl.ds(row, ZROWS)], sem).start()

        for c in (NCHUNK - 2, NCHUNK - 1):
            b, sem = bufs[c % 2], sems[c % 2]
            row = pl.multiple_of(base + c * ZROWS, ZROWS)
            pltpu.make_async_copy(b, out_hbm.at[pl.ds(row, ZROWS)], sem).wait()

    mesh = plsc.VectorSubcoreMesh(core_axis_name="c", subcore_axis_name="s")
    return pl.kernel(
        body,
        out_type=jax.ShapeDtypeStruct((N, C), jnp.float32),
        mesh=mesh,
        compiler_params=pltpu.CompilerParams(needs_layout_passes=False),
        scratch_types=[
            pltpu.VMEM((ZROWS, C), jnp.float32),
            pltpu.VMEM((ZROWS, C), jnp.float32),
            pltpu.VMEM((RPW,), jnp.int32),
            pltpu.VMEM((RPW,), jnp.float32),
            pltpu.SemaphoreType.DMA,
            pltpu.SemaphoreType.DMA,
        ],
    )


def kernel(grad_output, input, target, total_weight):
    N, C = input.shape
    tgt = target.astype(jnp.int32)
    return _make_sc_kernel(N, C)(grad_output, tgt)


# speedup vs baseline: 12.4821x; 1.0280x over previous
import jax
import jax.numpy as jnp
from jax import lax
from jax.experimental import pallas as pl
from jax.experimental.pallas import tpu as pltpu
from jax.experimental.pallas import tpu_sc as plsc

IGNORE = 1
L = 16
NC, NS = 2, 16
NW = NC * NS


def _make_sc_kernel(N, C):
    RPW = N // NW
    ZROWS = 4
    NCHUNK = RPW // ZROWS
    GRP = L // ZROWS

    def body(g_hbm, t_hbm, out_hbm, buf0, buf1, tloc, gloc, sem0, sem1):
        wid = lax.axis_index("s") * NC + lax.axis_index("c")
        base = wid * RPW

        pltpu.sync_copy(t_hbm.at[pl.ds(base, RPW)], tloc)
        pltpu.sync_copy(g_hbm.at[pl.ds(base, RPW)], gloc)

        zeros16 = jnp.zeros((L,), jnp.float32)
        iota16 = lax.iota(jnp.int32, L)
        UNROLL = 8

        def zinit(i, carry):
            for r in range(ZROWS):
                for u in range(UNROLL):
                    off = pl.multiple_of(i * (L * UNROLL), L * UNROLL) + u * L
                    buf0[r, pl.ds(off, L)] = zeros16
                    buf1[r, pl.ds(off, L)] = zeros16
            return carry

        lax.fori_loop(0, C // (L * UNROLL), zinit, 0)

        def chunk_vectors(c):
            grp = c // GRP
            sub = c % GRP
            t16 = tloc[pl.ds(grp * L, L)]
            g16 = gloc[pl.ds(grp * L, L)]
            val16 = jnp.where(t16 == IGNORE, zeros16, -g16)
            ridx16 = iota16 - ZROWS * sub
            mask16 = (iota16 >= ZROWS * sub) & (iota16 < ZROWS * (sub + 1))
            return t16, val16, ridx16, mask16

        def dma(b, sem, c):
            row = pl.multiple_of(base + c * ZROWS, ZROWS)
            return pltpu.make_async_copy(b, out_hbm.at[pl.ds(row, ZROWS)], sem)

        def do_chunk(c, b, sem):
            @pl.when(c >= 2)
            def _():
                dma(b, sem, c - 2).wait()
                pt16, _, pr16, pm16 = chunk_vectors(c - 2)
                plsc.store_scatter(b, [pr16, pt16], zeros16, mask=pm16)

            t16, val16, ridx16, mask16 = chunk_vectors(c)
            plsc.store_scatter(b, [ridx16, t16], val16, mask=mask16)
            dma(b, sem, c).start()

        def pair(p, carry):
            do_chunk(2 * p, buf0, sem0)
            do_chunk(2 * p + 1, buf1, sem1)
            return carry

        lax.fori_loop(0, NCHUNK // 2, pair, 0)

        dma(buf0, sem0, NCHUNK - 2).wait()
        dma(buf1, sem1, NCHUNK - 1).wait()

    mesh = plsc.VectorSubcoreMesh(core_axis_name="c", subcore_axis_name="s")
    return pl.kernel(
        body,
        out_type=jax.ShapeDtypeStruct((N, C), jnp.float32),
        mesh=mesh,
        compiler_params=pltpu.CompilerParams(needs_layout_passes=False),
        scratch_types=[
            pltpu.VMEM((ZROWS, C), jnp.float32),
            pltpu.VMEM((ZROWS, C), jnp.float32),
            pltpu.VMEM((RPW,), jnp.int32),
            pltpu.VMEM((RPW,), jnp.float32),
            pltpu.SemaphoreType.DMA,
            pltpu.SemaphoreType.DMA,
        ],
    )


def kernel(grad_output, input, target, total_weight):
    N, C = input.shape
    tgt = target.astype(jnp.int32)
    return _make_sc_kernel(N, C)(grad_output, tgt)
